# baseline (speedup 1.0000x reference)
import jax
import jax.numpy as jnp
from jax import lax
from jax.experimental import pallas as pl
from jax.experimental.pallas import tpu as pltpu
from jax.experimental.pallas import tpu_sc as plsc

VOCAB = 1000000
MAXLEN = 200
DIM = 64
BATCH = 4096

NC, NS = 2, 16
NW = NC * NS
TB = 128
NBLK = BATCH // TB
NTILE = MAXLEN * NBLK
TPW = NTILE // NW
NBUF = 4
TSTRIDE = TB + 5
VPR = DIM // 16


def _body(xt_hbm, tab_hbm, pos_hbm, out_hbm, idx_v, pos_v, rowbufs, tbufs,
          gsems, ssems):
    wid = lax.axis_index("s") * NC + lax.axis_index("c")
    t0 = wid * TPW

    pltpu.sync_copy(xt_hbm.at[pl.ds(t0, TPW)], idx_v)
    pltpu.sync_copy(pos_hbm, pos_v)

    iotav = lax.iota(jnp.int32, 16)

    def start_gather(g, b):
        pltpu.async_copy(tab_hbm.at[idx_v.at[g]], rowbufs.at[b], gsems[b])

    def wait_gather(g, b):
        pltpu.make_async_copy(tab_hbm.at[idx_v.at[g]], rowbufs.at[b],
                              gsems[b]).wait()

    def out_slice(g):
        t = t0 + g
        return out_hbm.at[t // NBLK, :, t % NBLK]

    def start_scatter(g, b):
        pltpu.async_copy(tbufs.at[b, :, :, pl.ds(0, TB)], out_slice(g), ssems[b])

    def wait_scatter(g, b):
        pltpu.make_async_copy(tbufs.at[b, :, :, pl.ds(0, TB)], out_slice(g),
                              ssems[b]).wait()

    for b in range(NBUF - 1):
        start_gather(b, b)

    @pl.loop(0, TPW // NBUF)
    def _outer(o):
        g0 = o * NBUF
        for k in range(NBUF):
            g = g0 + k

            @pl.when(g + NBUF - 1 < TPW)
            def _issue():
                start_gather(g + NBUF - 1, (k + NBUF - 1) % NBUF)

            wait_gather(g, k)

            @pl.when(g >= NBUF)
            def _recycle():
                wait_scatter(g - NBUF, k)

            p = (t0 + g) // NBLK
            pvecs = [pos_v[p, pl.ds(j * 16, 16)] for j in range(VPR)]
            rows = [iotav + j * 16 for j in range(VPR)]
            rows_cg = [r // 8 for r in rows]
            rows_cr = [r % 8 for r in rows]

            @plsc.parallel_loop(0, TB, unroll=8)
            def _transpose(r):
                rb = jnp.full((16,), r, jnp.int32)
                for j in range(VPR):
                    v = rowbufs[k, r, pl.ds(j * 16, 16)] + pvecs[j]
                    plsc.store_scatter(tbufs.at[k],
                                       [rows_cg[j], rows_cr[j], rb], v)

            start_scatter(g, k)

    for k in range(NBUF):
        g = TPW - NBUF + k
        wait_scatter(g, g % NBUF)


@jax.jit
def kernel(x, token_table, pos_table):
    xt = x.T.reshape(NTILE, TB).astype(jnp.int32)
    mesh = plsc.VectorSubcoreMesh(core_axis_name="c", subcore_axis_name="s")
    fn = pl.kernel(
        _body,
        out_type=jax.ShapeDtypeStruct((MAXLEN, DIM // 8, BATCH // TB, 8, TB),
                                      jnp.float32),
        mesh=mesh,
        compiler_params=pltpu.CompilerParams(use_tc_tiling_on_sc=False,
                                             needs_layout_passes=False),
        scratch_types=[
            pltpu.VMEM((TPW, TB), jnp.int32),
            pltpu.VMEM((MAXLEN, DIM), jnp.float32),
            pltpu.VMEM((NBUF, TB, DIM), jnp.float32),
            pltpu.VMEM((NBUF, DIM // 8, 8, TSTRIDE), jnp.float32),
            [pltpu.SemaphoreType.DMA] * NBUF,
            [pltpu.SemaphoreType.DMA] * NBUF,
        ],
    )
    out = fn(xt, token_table, pos_table)
    return out.transpose(2, 4, 0, 1, 3).reshape(BATCH, MAXLEN, DIM)


# speedup vs baseline: 1.0194x; 1.0194x over previous
import jax
import jax.numpy as jnp
from jax import lax
from jax.experimental import pallas as pl
from jax.experimental.pallas import tpu as pltpu
from jax.experimental.pallas import tpu_sc as plsc

VOCAB = 1000000
MAXLEN = 200
DIM = 64
BATCH = 4096

NC, NS = 2, 16
NW = NC * NS
TB = 128
NBLK = BATCH // TB
NTILE = MAXLEN * NBLK
TPW = NTILE // NW
NBUF = 4
TSTRIDE = TB + 5
VPR = DIM // 16


def _body(xt_hbm, tab_hbm, pos_hbm, out_hbm, idx_v, pos_v, rowbufs, tbufs,
          gsems, ssems):
    wid = lax.axis_index("s") * NC + lax.axis_index("c")
    t0 = wid * TPW

    pltpu.sync_copy(xt_hbm.at[pl.ds(t0, TPW)], idx_v)
    pltpu.sync_copy(pos_hbm, pos_v)

    iotav = lax.iota(jnp.int32, 16)

    def start_gather(g, b):
        pltpu.async_copy(tab_hbm.at[idx_v.at[g]], rowbufs.at[b], gsems[b])

    def wait_gather(g, b):
        pltpu.make_async_copy(tab_hbm.at[idx_v.at[g]], rowbufs.at[b],
                              gsems[b]).wait()

    def out_slice(g):
        t = t0 + g
        return out_hbm.at[t // NBLK, :, t % NBLK]

    def start_scatter(g, b):
        pltpu.async_copy(tbufs.at[b, :, :, pl.ds(0, TB)], out_slice(g), ssems[b])

    def wait_scatter(g, b):
        pltpu.make_async_copy(tbufs.at[b, :, :, pl.ds(0, TB)], out_slice(g),
                              ssems[b]).wait()

    for b in range(NBUF - 1):
        start_gather(b, b)

    @pl.loop(0, TPW // NBUF)
    def _outer(o):
        g0 = o * NBUF
        for k in range(NBUF):
            g = g0 + k

            @pl.when(g + NBUF - 1 < TPW)
            def _issue():
                start_gather(g + NBUF - 1, (k + NBUF - 1) % NBUF)

            wait_gather(g, k)

            @pl.when(g >= NBUF)
            def _recycle():
                wait_scatter(g - NBUF, k)

            p = (t0 + g) // NBLK
            pvecs = [pos_v[p, pl.ds(j * 16, 16)] for j in range(VPR)]
            rows = [iotav + j * 16 for j in range(VPR)]
            rows_cg = [r // 8 for r in rows]
            rows_cr = [r % 8 for r in rows]

            @plsc.parallel_loop(0, TB, unroll=4)
            def _transpose(r):
                rb = jnp.full((16,), r, jnp.int32)
                for j in range(VPR):
                    v = rowbufs[k, r, pl.ds(j * 16, 16)] + pvecs[j]
                    plsc.store_scatter(tbufs.at[k],
                                       [rows_cg[j], rows_cr[j], rb], v)

            start_scatter(g, k)

    for k in range(NBUF):
        g = TPW - NBUF + k
        wait_scatter(g, g % NBUF)


@jax.jit
def kernel(x, token_table, pos_table):
    xt = x.T.reshape(NTILE, TB).astype(jnp.int32)
    mesh = plsc.VectorSubcoreMesh(core_axis_name="c", subcore_axis_name="s")
    fn = pl.kernel(
        _body,
        out_type=jax.ShapeDtypeStruct((MAXLEN, DIM // 8, BATCH // TB, 8, TB),
                                      jnp.float32),
        mesh=mesh,
        compiler_params=pltpu.CompilerParams(use_tc_tiling_on_sc=False,
                                             needs_layout_passes=False),
        scratch_types=[
            pltpu.VMEM((TPW, TB), jnp.int32),
            pltpu.VMEM((MAXLEN, DIM), jnp.float32),
            pltpu.VMEM((NBUF, TB, DIM), jnp.float32),
            pltpu.VMEM((NBUF, DIM // 8, 8, TSTRIDE), jnp.float32),
            [pltpu.SemaphoreType.DMA] * NBUF,
            [pltpu.SemaphoreType.DMA] * NBUF,
        ],
    )
    out = fn(xt, token_table, pos_table)
    return out.transpose(2, 4, 0, 1, 3).reshape(BATCH, MAXLEN, DIM)


# speedup vs baseline: 1.0195x; 1.0001x over previous
import jax
import jax.numpy as jnp
from jax import lax
from jax.experimental import pallas as pl
from jax.experimental.pallas import tpu as pltpu
from jax.experimental.pallas import tpu_sc as plsc

VOCAB = 1000000
MAXLEN = 200
DIM = 64
BATCH = 4096

NC, NS = 2, 16
NW = NC * NS
TB = 128
NBLK = BATCH // TB
NTILE = MAXLEN * NBLK
TPW = NTILE // NW
NBUF = 4
TSTRIDE = TB + 5
VPR = DIM // 16


def _body(xt_hbm, tab_hbm, pos_hbm, out_hbm, idx_v, pos_v, rowbufs, tbufs,
          gsems, ssems):
    wid = lax.axis_index("s") * NC + lax.axis_index("c")
    t0 = wid * TPW

    pltpu.sync_copy(xt_hbm.at[pl.ds(t0, TPW)], idx_v)
    pltpu.sync_copy(pos_hbm, pos_v)

    iotav = lax.iota(jnp.int32, 16)

    def start_gather(g, b):
        pltpu.async_copy(tab_hbm.at[idx_v.at[g]], rowbufs.at[b], gsems[b])

    def wait_gather(g, b):
        pltpu.make_async_copy(tab_hbm.at[idx_v.at[g]], rowbufs.at[b],
                              gsems[b]).wait()

    def out_slice(g):
        t = t0 + g
        return out_hbm.at[t // NBLK, :, t % NBLK]

    def start_scatter(g, b):
        pltpu.async_copy(tbufs.at[b, :, :, pl.ds(0, TB)], out_slice(g), ssems[b])

    def wait_scatter(g, b):
        pltpu.make_async_copy(tbufs.at[b, :, :, pl.ds(0, TB)], out_slice(g),
                              ssems[b]).wait()

    for b in range(NBUF - 1):
        start_gather(b, b)

    @pl.loop(0, TPW // NBUF)
    def _outer(o):
        g0 = o * NBUF
        for k in range(NBUF):
            g = g0 + k

            @pl.when(g >= 1)
            def _flush():
                start_scatter(g - 1, (k + NBUF - 1) % NBUF)

            @pl.when(g + NBUF - 1 < TPW)
            def _issue():
                start_gather(g + NBUF - 1, (k + NBUF - 1) % NBUF)

            wait_gather(g, k)

            @pl.when(g >= NBUF)
            def _recycle():
                wait_scatter(g - NBUF, k)

            p = (t0 + g) // NBLK
            pvecs = [pos_v[p, pl.ds(j * 16, 16)] for j in range(VPR)]
            rows = [iotav + j * 16 for j in range(VPR)]
            rows_cg = [r // 8 for r in rows]
            rows_cr = [r % 8 for r in rows]

            @plsc.parallel_loop(0, TB, unroll=4)
            def _transpose(r):
                rb = jnp.full((16,), r, jnp.int32)
                for j in range(VPR):
                    v = rowbufs[k, r, pl.ds(j * 16, 16)] + pvecs[j]
                    plsc.store_scatter(tbufs.at[k],
                                       [rows_cg[j], rows_cr[j], rb], v)

    start_scatter(TPW - 1, (TPW - 1) % NBUF)
    for k in range(NBUF):
        g = TPW - NBUF + k
        wait_scatter(g, g % NBUF)


@jax.jit
def kernel(x, token_table, pos_table):
    xt = x.T.reshape(NTILE, TB).astype(jnp.int32)
    mesh = plsc.VectorSubcoreMesh(core_axis_name="c", subcore_axis_name="s")
    fn = pl.kernel(
        _body,
        out_type=jax.ShapeDtypeStruct((MAXLEN, DIM // 8, BATCH // TB, 8, TB),
                                      jnp.float32),
        mesh=mesh,
        compiler_params=pltpu.CompilerParams(use_tc_tiling_on_sc=False,
                                             needs_layout_passes=False),
        scratch_types=[
            pltpu.VMEM((TPW, TB), jnp.int32),
            pltpu.VMEM((MAXLEN, DIM), jnp.float32),
            pltpu.VMEM((NBUF, TB, DIM), jnp.float32),
            pltpu.VMEM((NBUF, DIM // 8, 8, TSTRIDE), jnp.float32),
            [pltpu.SemaphoreType.DMA] * NBUF,
            [pltpu.SemaphoreType.DMA] * NBUF,
        ],
    )
    out = fn(xt, token_table, pos_table)
    return out.transpose(2, 4, 0, 1, 3).reshape(BATCH, MAXLEN, DIM)
